# baseline (speedup 1.0000x reference)
import functools

import jax
import jax.numpy as jnp
from jax import lax
from jax.experimental import pallas as pl
from jax.experimental.pallas import tpu as pltpu
from jax.experimental.pallas import tpu_sc as plsc

_NC = 2
_NS = 16
_NW = _NC * _NS
_L = 16


def _fea2_sparsecore(r, edge_index):
    n_nodes = r.shape[0]
    e = edge_index.shape[1]
    chunk = 3584
    n_full = e // chunk
    rem = e - n_full * chunk
    assert rem % 128 == 0 and chunk % _L == 0
    max_mine = (n_full + _NW - 1) // _NW
    n_pairs = (max_mine + 1) // 2

    mesh = plsc.VectorSubcoreMesh(
        core_axis_name="c", subcore_axis_name="s",
        num_cores=_NC, num_subcores=_NS)

    @functools.partial(
        pl.kernel,
        mesh=mesh,
        out_type=jax.ShapeDtypeStruct((2, e), jnp.float32),
        compiler_params=pltpu.CompilerParams(needs_layout_passes=False),
        scratch_types=[
            pltpu.VMEM((n_nodes,), jnp.float32),
            pltpu.VMEM((2, chunk), jnp.int32),
            pltpu.VMEM((2, chunk), jnp.int32),
            pltpu.VMEM((2, chunk), jnp.float32),
            pltpu.VMEM((2, chunk), jnp.float32),
            pltpu.SemaphoreType.DMA,
            pltpu.SemaphoreType.DMA,
            pltpu.SemaphoreType.DMA,
            pltpu.SemaphoreType.DMA,
            pltpu.SemaphoreType.DMA,
        ],
    )
    def k(r_hbm, ei_hbm, out_hbm, r_v, i0, i1, o0, o1, sr, si0, si1, so0, so1):
        wid = lax.axis_index("s") * _NC + lax.axis_index("c")
        ibufs, obufs = (i0, i1), (o0, o1)
        isems, osems = (si0, si1), (so0, so1)
        n_mine = (n_full - wid + _NW - 1) // _NW

        def in_start(s, b):
            pltpu.async_copy(
                ei_hbm.at[:, pl.ds((wid + s * _NW) * chunk, chunk)],
                ibufs[b], isems[b])

        def compute(ib, ob, n):
            @plsc.parallel_loop(0, n, step=_L, unroll=8)
            def body(off):
                idx_r = ib[0, pl.ds(off, _L)]
                idx_c = ib[1, pl.ds(off, _L)]
                ob[0, pl.ds(off, _L)] = plsc.load_gather(r_v, [idx_r])
                ob[1, pl.ds(off, _L)] = plsc.load_gather(r_v, [idx_c])

        rcp = pltpu.async_copy(r_hbm, r_v, sr)
        in_start(0, 0)
        in_start(1, 1)
        rcp.wait()

        def pair(g, _):
            for b in (0, 1):
                s = 2 * g + b

                @pl.when(s < n_mine)
                def _():
                    pltpu.make_async_copy(
                        ei_hbm.at[:, pl.ds(0, chunk)], ibufs[b], isems[b]).wait()

                    @pl.when(g > 0)
                    def _():
                        pltpu.make_async_copy(
                            obufs[b], out_hbm.at[:, pl.ds(0, chunk)],
                            osems[b]).wait()

                    compute(ibufs[b], obufs[b], chunk)
                    pltpu.async_copy(
                        obufs[b],
                        out_hbm.at[:, pl.ds((wid + s * _NW) * chunk, chunk)],
                        osems[b])

                    @pl.when(s + 2 < n_mine)
                    def _():
                        in_start(s + 2, b)
            return 0

        lax.fori_loop(0, n_pairs, pair, 0)

        for b in (0, 1):
            pltpu.make_async_copy(
                obufs[b], out_hbm.at[:, pl.ds(0, chunk)], osems[b]).wait()

        if rem:
            @pl.when(wid == 30)
            def _():
                base = n_full * chunk
                pltpu.sync_copy(ei_hbm.at[:, pl.ds(base, rem)],
                                i0.at[:, pl.ds(0, rem)])
                compute(i0, o0, rem)
                pltpu.sync_copy(o0.at[:, pl.ds(0, rem)],
                                out_hbm.at[:, pl.ds(base, rem)])

    return k(r, edge_index)


def _cos_poly(v):
    two_over_pi = 0.6366197723675814
    p1 = 1.5703125
    p2 = 4.837512969970703125e-4
    p3 = 7.54978995489188608e-8
    kf = jnp.floor(v * two_over_pi + 0.5)
    y = ((v - kf * p1) - kf * p2) - kf * p3
    ki = kf.astype(jnp.int32)
    z = y * y
    cosp = 1.0 + z * (-0.5 + z * (4.166664568298827e-2
                                  + z * (-1.388731625493765e-3
                                         + z * 2.443315711809948e-5)))
    sinp = y + y * z * (-1.6666654611e-1
                        + z * (8.3321608736e-3 + z * (-1.9515295891e-4)))
    res = jnp.where((ki & 1) == 1, sinp, cosp)
    return jnp.where(((ki + 1) & 2) != 0, -res, res)


def _fea1_tensorcore(ht):
    d, e = ht.shape
    block_cols = 64000
    assert e % block_cols == 0

    def body(x_ref, o_ref):
        v = x_ref[...]
        sub = lax.broadcasted_iota(jnp.int32, v.shape, 0)
        o_ref[...] = jnp.where(sub == 0, v, _cos_poly(v))

    return pl.pallas_call(
        body,
        grid=(e // block_cols,),
        in_specs=[pl.BlockSpec((d, block_cols), lambda i: (0, i))],
        out_specs=pl.BlockSpec((d, block_cols), lambda i: (0, i)),
        out_shape=jax.ShapeDtypeStruct((d, e), jnp.float32),
    )(ht)


def kernel(r, h, edge_index):
    fea2 = _fea2_sparsecore(r, edge_index.astype(jnp.int32)).T
    fea1 = _fea1_tensorcore(h.T).T
    return (fea1, fea2)
